# baseline (speedup 1.0000x reference)
import dataclasses
import functools

import jax
import jax.numpy as jnp
from jax import lax
from jax.experimental import pallas as pl
from jax.experimental.pallas import tpu as pltpu
from jax.experimental.pallas import tpu_sc as plsc

N = 100000
E = 1600000
EP = 1638400
PAD = EP - E
D = 32
H = 16
NC = 2
NS = 16
NSINK = 128
NA = N + NSINK
IR = 128
CH = 8
KE = CH * IR
LCH = 4
LKE = LCH * IR
DR = EP // IR
ER = E // IR
PR = PAD // IR
ZB = 1024
ZT = NA // NS
OT = N // NS
F32 = jnp.float32
I32 = jnp.int32


def _mesh():
    return plsc.VectorSubcoreMesh(core_axis_name="c", subcore_axis_name="s")


def _sc_params(layout_passes=True):
    cp = pltpu.CompilerParams()
    fields = getattr(pltpu.CompilerParams, "__dataclass_fields__", {})
    if "use_tc_tiling_on_sc" in fields:
        cp = dataclasses.replace(cp, use_tc_tiling_on_sc=False)
    if not layout_passes and "needs_layout_passes" in fields:
        cp = dataclasses.replace(cp, needs_layout_passes=False)
    return cp


def _zero_acc(zbuf, zb_rows, acc_sh, s):
    @pl.loop(0, zb_rows)
    def _(j):
        zbuf[j, :] = jnp.zeros((H,), F32)

    offs = list(range(0, ZT - zb_rows, zb_rows)) + [ZT - zb_rows]
    for off in offs:
        pltpu.sync_copy(zbuf.at[pl.ds(0, zb_rows)],
                        acc_sh.at[pl.ds(s * ZT + off, zb_rows)])


def _sc_degree(colm, cpad):
    @functools.partial(
        pl.kernel,
        out_type=jax.ShapeDtypeStruct((NC, N, H), F32),
        mesh=_mesh(),
        scratch_types=[
            pltpu.VMEM((CH, IR), I32),
            pltpu.VMEM((KE, H), F32),
            pltpu.VMEM_SHARED((NA, H), F32),
        ],
        compiler_params=_sc_params(),
    )
    def k(col_hbm, cpad_hbm, deg_hbm, ci, ones_v, acc_sh):
        c = lax.axis_index("c")
        s = lax.axis_index("s")
        _zero_acc(ones_v, ZB, acc_sh, s)
        plsc.subcore_barrier()

        @pl.loop(0, KE)
        def _(j):
            ones_v[j, :] = jnp.ones((H,), F32)

        rows_w = DR // (NC * NS)
        base = (c * NS + s) * rows_w
        HCH = CH // 2

        @pl.loop(0, rows_w, step=CH)
        def _(r):
            g = base + r

            @pl.when(g + CH <= ER)
            def _():
                pltpu.sync_copy(col_hbm.at[pl.ds(g, CH)], ci)

            @pl.when(g >= ER)
            def _():
                pltpu.sync_copy(cpad_hbm.at[pl.ds(g - ER, CH)], ci)

            @pl.when(jnp.logical_and(g < ER, g + CH > ER))
            def _():
                pltpu.sync_copy(col_hbm.at[pl.ds(g, HCH)],
                                ci.at[pl.ds(0, HCH)])
                pltpu.sync_copy(cpad_hbm.at[pl.ds(0, HCH)],
                                ci.at[pl.ds(HCH, HCH)])

            for i in range(CH):
                pltpu.sync_copy(ones_v.at[pl.ds(0, IR)],
                                acc_sh.at[ci.at[i]], add=True)

        plsc.subcore_barrier()
        pltpu.sync_copy(acc_sh.at[pl.ds(s * OT, OT)],
                        deg_hbm.at[c].at[pl.ds(s * OT, OT)])

    return k(colm, cpad)


def _sc_layer(rowm, rpad, colm, cpad, yflat):
    @functools.partial(
        pl.kernel,
        out_type=jax.ShapeDtypeStruct((NC * N, H), F32),
        mesh=_mesh(),
        scratch_types=[
            pltpu.VMEM((LCH, IR), I32),
            pltpu.VMEM((LCH, IR), I32),
            pltpu.VMEM((LCH, IR), I32),
            pltpu.VMEM((LCH, IR), I32),
            pltpu.VMEM((LKE, H), F32),
            pltpu.VMEM((LKE, H), F32),
            pltpu.VMEM_SHARED((NA, H), F32),
            pltpu.SemaphoreType.DMA,
            pltpu.SemaphoreType.DMA,
            pltpu.SemaphoreType.DMA,
        ],
        compiler_params=_sc_params(),
    )
    def k(row_hbm, rpad_hbm, col_hbm, cpad_hbm, y_hbm, s_hbm,
          ri0, ci0, ri1, ci1, rv0, rv1, acc_sh, sem_i, sem_g, sem_s):
        c = lax.axis_index("c")
        s = lax.axis_index("s")
        _zero_acc(rv0, LKE, acc_sh, s)
        plsc.subcore_barrier()

        rows_t = DR // NS
        base = s * rows_t
        off0 = c * N

        def drain_scatters(ci_, rv_):
            for i in range(LCH):
                pltpu.make_async_copy(rv_.at[pl.ds(i * IR, IR)],
                                      acc_sh.at[ci_.at[i]], sem_s).wait()

        def chunk(r, ri_, ci_, rv_):
            @pl.when(r >= 2 * LCH)
            def _():
                drain_scatters(ci_, rv_)

            g = base + r

            @pl.when(g + LCH <= ER)
            def _():
                pltpu.sync_copy(row_hbm.at[pl.ds(g, LCH)], ri_)
                pltpu.sync_copy(col_hbm.at[pl.ds(g, LCH)], ci_)

            @pl.when(g + LCH > ER)
            def _():
                pltpu.sync_copy(rpad_hbm.at[pl.ds(g - ER, LCH)], ri_)
                pltpu.sync_copy(cpad_hbm.at[pl.ds(g - ER, LCH)], ci_)

            @pl.loop(0, LCH)
            def _(i):
                @pl.loop(0, IR, step=16)
                def _(l):
                    ri_[i, pl.ds(l, 16)] = ri_[i, pl.ds(l, 16)] + off0

            for i in range(LCH):
                pltpu.async_copy(y_hbm.at[ri_.at[i]],
                                 rv_.at[pl.ds(i * IR, IR)], sem_g)
            for i in range(LCH):
                pltpu.make_async_copy(y_hbm.at[ri_.at[i]],
                                      rv_.at[pl.ds(i * IR, IR)],
                                      sem_g).wait()
            for i in range(LCH):
                pltpu.async_copy(rv_.at[pl.ds(i * IR, IR)],
                                 acc_sh.at[ci_.at[i]], sem_s, add=True)

        @pl.loop(0, rows_t, step=2 * LCH)
        def _(r):
            chunk(r, ri0, ci0, rv0)
            chunk(r + LCH, ri1, ci1, rv1)

        drain_scatters(ci0, rv0)
        drain_scatters(ci1, rv1)
        plsc.subcore_barrier()
        pltpu.sync_copy(acc_sh.at[pl.ds(s * OT, OT)],
                        s_hbm.at[pl.ds(c * N + s * OT, OT)])

    return k(rowm, rpad, colm, cpad, yflat)


SCH = 4
SKE = SCH * IR


def _sc_score(rowm, colm, pad, tbl):
    @functools.partial(
        pl.kernel,
        out_type=jax.ShapeDtypeStruct((EP,), F32),
        mesh=_mesh(),
        scratch_types=[
            pltpu.VMEM((SCH, IR), I32),
            pltpu.VMEM((SCH, IR), I32),
            pltpu.VMEM((SCH, IR), I32),
            pltpu.VMEM((SCH, IR), I32),
            pltpu.VMEM((SKE, D), F32),
            pltpu.VMEM((SKE, D), F32),
            pltpu.VMEM((SKE, D), F32),
            pltpu.VMEM((SKE, D), F32),
            pltpu.VMEM((SKE,), F32),
            pltpu.VMEM((SKE,), F32),
            pltpu.SemaphoreType.DMA,
            pltpu.SemaphoreType.DMA,
            pltpu.SemaphoreType.DMA,
        ],
        compiler_params=_sc_params(layout_passes=False),
    )
    def k(row_hbm, col_hbm, pad_hbm, tbl_hbm, out_hbm, ri0, ci0, ri1, ci1,
          av0, bv0, av1, bv1, sv0, sv1, sem_i, sem_g, sem_o):
        c = lax.axis_index("c")
        s = lax.axis_index("s")
        lane = lax.iota(I32, 16)
        rows_w = DR // (NC * NS)
        base = (s * NC + c) * rows_w

        def fire(r, ri_, ci_, av_, bv_):
            g = base + r

            @pl.when(g + SCH <= ER)
            def _():
                pltpu.sync_copy(row_hbm.at[pl.ds(g, SCH)], ri_)
                pltpu.sync_copy(col_hbm.at[pl.ds(g, SCH)], ci_)

            @pl.when(g + SCH > ER)
            def _():
                pltpu.sync_copy(pad_hbm.at[pl.ds(g - ER, SCH)], ri_)
                pltpu.sync_copy(pad_hbm.at[pl.ds(g - ER, SCH)], ci_)
            for i in range(SCH):
                pltpu.async_copy(tbl_hbm.at[ri_.at[i]],
                                 av_.at[pl.ds(i * IR, IR)], sem_g)
                pltpu.async_copy(tbl_hbm.at[ci_.at[i]],
                                 bv_.at[pl.ds(i * IR, IR)], sem_g)

        def process(r, ri_, ci_, av_, bv_, sv_):
            for i in range(SCH):
                pltpu.make_async_copy(tbl_hbm.at[ri_.at[i]],
                                      av_.at[pl.ds(i * IR, IR)], sem_g).wait()
                pltpu.make_async_copy(tbl_hbm.at[ci_.at[i]],
                                      bv_.at[pl.ds(i * IR, IR)], sem_g).wait()

            @pl.when(r >= 2 * SCH)
            def _():
                pltpu.make_async_copy(
                    sv_, out_hbm.at[pl.ds((base + r) * IR, SKE)], sem_o).wait()

            @pl.loop(0, SKE, step=16)
            def _(j0):
                ridx = lane + j0
                acc = jnp.zeros((16,), F32)
                for d in range(D):
                    cidx = (lane + d) & (D - 1)
                    ga = plsc.load_gather(av_, [ridx, cidx])
                    gb = plsc.load_gather(bv_, [ridx, cidx])
                    acc = acc + ga * gb
                sv_[pl.ds(j0, 16)] = acc

            pltpu.async_copy(sv_, out_hbm.at[pl.ds((base + r) * IR, SKE)],
                             sem_o)

        fire(0, ri0, ci0, av0, bv0)

        @pl.loop(0, rows_w, step=2 * SCH)
        def _(r):
            fire(r + SCH, ri1, ci1, av1, bv1)
            process(r, ri0, ci0, av0, bv0, sv0)

            @pl.when(r + 2 * SCH < rows_w)
            def _():
                fire(r + 2 * SCH, ri0, ci0, av0, bv0)

            process(r + SCH, ri1, ci1, av1, bv1, sv1)

        pltpu.make_async_copy(sv0, out_hbm.at[pl.ds(base * IR, SKE)],
                              sem_o).wait()
        pltpu.make_async_copy(sv1, out_hbm.at[pl.ds(base * IR, SKE)],
                              sem_o).wait()

    return k(rowm, colm, pad, tbl)


WR = N // (NC * NS)


def _offs(total, chb):
    return list(range(0, total - chb, chb)) + [total - chb]


def _sc_prep(degp, emb):
    CHB = 1024

    @functools.partial(
        pl.kernel,
        out_type=[
            jax.ShapeDtypeStruct((N, H), F32),
            jax.ShapeDtypeStruct((NC * N, H), F32),
        ],
        mesh=_mesh(),
        scratch_types=[
            pltpu.VMEM((CHB, H), F32),
            pltpu.VMEM((CHB, H), F32),
            pltpu.VMEM((CHB, D), F32),
            pltpu.VMEM((CHB, H), F32),
            pltpu.VMEM((CHB, H), F32),
            pltpu.VMEM((CHB, H), F32),
        ],
        compiler_params=_sc_params(),
    )
    def k(degp_hbm, emb_hbm, dis_hbm, y_hbm, d0v, d1v, ev, dv, y0v, y1v):
        c = lax.axis_index("c")
        s = lax.axis_index("s")
        w0 = (c * NS + s) * WR

        for off in _offs(WR, CHB):
            row0 = w0 + off
            pltpu.sync_copy(degp_hbm.at[0].at[pl.ds(row0, CHB)], d0v)
            pltpu.sync_copy(degp_hbm.at[1].at[pl.ds(row0, CHB)], d1v)
            pltpu.sync_copy(emb_hbm.at[pl.ds(row0, CHB)], ev)

            @pl.loop(0, CHB)
            def _(j):
                deg = d0v[j, :] + d1v[j, :]
                xi = lax.bitcast_convert_type(deg, I32)
                yi = jnp.int32(0x5F3759DF) - lax.shift_right_logical(
                    xi, jnp.ones_like(xi))
                y = lax.bitcast_convert_type(yi, F32)
                h = deg * jnp.float32(0.5)
                for _ in range(4):
                    y = y * (jnp.float32(1.5) - h * y * y)
                dis = jnp.where(deg > 0.0, y, 0.0)
                dv[j, :] = dis
                y0v[j, :] = dis * ev[j, pl.ds(0, H)]
                y1v[j, :] = dis * ev[j, pl.ds(H, H)]

            pltpu.sync_copy(dv, dis_hbm.at[pl.ds(row0, CHB)])
            pltpu.sync_copy(y0v, y_hbm.at[pl.ds(row0, CHB)])
            pltpu.sync_copy(y1v, y_hbm.at[pl.ds(N + row0, CHB)])

    return k(degp, emb)


def _sc_mid(s1, dis16):
    CHB = 1024

    @functools.partial(
        pl.kernel,
        out_type=[
            jax.ShapeDtypeStruct((NC * N, H), F32),
            jax.ShapeDtypeStruct((NC * N, H), F32),
        ],
        mesh=_mesh(),
        scratch_types=[
            pltpu.VMEM((CHB, H), F32),
            pltpu.VMEM((CHB, H), F32),
            pltpu.VMEM((CHB, H), F32),
            pltpu.VMEM((CHB, H), F32),
            pltpu.VMEM((CHB, H), F32),
            pltpu.VMEM((CHB, H), F32),
            pltpu.VMEM((CHB, H), F32),
        ],
        compiler_params=_sc_params(),
    )
    def k(s_hbm, dis_hbm, x_hbm, y_hbm, s0v, s1v, dv, x0v, x1v, y0v, y1v):
        c = lax.axis_index("c")
        s = lax.axis_index("s")
        w0 = (c * NS + s) * WR

        for off in _offs(WR, CHB):
            row0 = w0 + off
            pltpu.sync_copy(s_hbm.at[pl.ds(row0, CHB)], s0v)
            pltpu.sync_copy(s_hbm.at[pl.ds(N + row0, CHB)], s1v)
            pltpu.sync_copy(dis_hbm.at[pl.ds(row0, CHB)], dv)

            @pl.loop(0, CHB)
            def _(j):
                dis = dv[j, :]
                xa = dis * s0v[j, :]
                xb = dis * s1v[j, :]
                x0v[j, :] = xa
                x1v[j, :] = xb
                y0v[j, :] = dis * xa
                y1v[j, :] = dis * xb

            pltpu.sync_copy(x0v, x_hbm.at[pl.ds(row0, CHB)])
            pltpu.sync_copy(x1v, x_hbm.at[pl.ds(N + row0, CHB)])
            pltpu.sync_copy(y0v, y_hbm.at[pl.ds(row0, CHB)])
            pltpu.sync_copy(y1v, y_hbm.at[pl.ds(N + row0, CHB)])

    return k(s1, dis16)


def _sc_final(emb, x1, s2, dis16):
    CHB = 768

    @functools.partial(
        pl.kernel,
        out_type=jax.ShapeDtypeStruct((N, D), F32),
        mesh=_mesh(),
        scratch_types=[
            pltpu.VMEM((CHB, D), F32),
            pltpu.VMEM((CHB, H), F32),
            pltpu.VMEM((CHB, H), F32),
            pltpu.VMEM((CHB, H), F32),
            pltpu.VMEM((CHB, H), F32),
            pltpu.VMEM((CHB, H), F32),
            pltpu.VMEM((CHB, D), F32),
        ],
        compiler_params=_sc_params(),
    )
    def k(emb_hbm, x_hbm, s_hbm, dis_hbm, tbl_hbm,
          ev, x0v, x1v, s0v, s1v, dv, ov):
        c = lax.axis_index("c")
        s = lax.axis_index("s")
        w0 = (c * NS + s) * WR
        athird = jnp.float32(1.0 / 3.0)

        for off in _offs(WR, CHB):
            row0 = w0 + off
            pltpu.sync_copy(emb_hbm.at[pl.ds(row0, CHB)], ev)
            pltpu.sync_copy(x_hbm.at[pl.ds(row0, CHB)], x0v)
            pltpu.sync_copy(x_hbm.at[pl.ds(N + row0, CHB)], x1v)
            pltpu.sync_copy(s_hbm.at[pl.ds(row0, CHB)], s0v)
            pltpu.sync_copy(s_hbm.at[pl.ds(N + row0, CHB)], s1v)
            pltpu.sync_copy(dis_hbm.at[pl.ds(row0, CHB)], dv)

            @pl.loop(0, CHB)
            def _(j):
                dis = dv[j, :]
                ov[j, pl.ds(0, H)] = (ev[j, pl.ds(0, H)] + x0v[j, :]
                                      + dis * s0v[j, :]) * athird
                ov[j, pl.ds(H, H)] = (ev[j, pl.ds(H, H)] + x1v[j, :]
                                      + dis * s1v[j, :]) * athird

            pltpu.sync_copy(ov, tbl_hbm.at[pl.ds(row0, CHB)])

    return k(emb, x1, s2, dis16)


def kernel(edge_index, emb):
    rowm = edge_index[0].reshape(ER, IR)
    colm = edge_index[1].reshape(ER, IR)
    pad = jnp.arange(PAD, dtype=I32)
    padg = (pad % N).reshape(PR, IR)
    pads = (N + (pad % NSINK)).reshape(PR, IR)

    degp = _sc_degree(colm, pads)
    dis16, y1 = _sc_prep(degp, emb)
    s1 = _sc_layer(rowm, padg, colm, pads, y1)
    x1, y2 = _sc_mid(s1, dis16)
    s2 = _sc_layer(rowm, padg, colm, pads, y2)
    out_tbl = _sc_final(emb, x1, s2, dis16)
    return _sc_score(rowm, colm, padg, out_tbl)[:E]


# speedup vs baseline: 1.1671x; 1.1671x over previous
import dataclasses
import functools

import jax
import jax.numpy as jnp
from jax import lax
from jax.experimental import pallas as pl
from jax.experimental.pallas import tpu as pltpu
from jax.experimental.pallas import tpu_sc as plsc

N = 100000
E = 1600000
EP = 1638400
PAD = EP - E
D = 32
H = 16
NC = 2
NS = 16
NSINK = 128
NA = N + NSINK
IR = 128
CH = 8
KE = CH * IR
LCH = 4
LKE = LCH * IR
DR = EP // IR
ER = E // IR
PR = PAD // IR
ZB = 1024
ZT = NA // NS
OT = N // NS
F32 = jnp.float32
I32 = jnp.int32


def _mesh():
    return plsc.VectorSubcoreMesh(core_axis_name="c", subcore_axis_name="s")


def _sc_params(layout_passes=True):
    cp = pltpu.CompilerParams()
    fields = getattr(pltpu.CompilerParams, "__dataclass_fields__", {})
    if "use_tc_tiling_on_sc" in fields:
        cp = dataclasses.replace(cp, use_tc_tiling_on_sc=False)
    if not layout_passes and "needs_layout_passes" in fields:
        cp = dataclasses.replace(cp, needs_layout_passes=False)
    return cp


def _zero_acc(zbuf, zb_rows, acc_sh, s):
    @pl.loop(0, zb_rows)
    def _(j):
        zbuf[j, :] = jnp.zeros((H,), F32)

    offs = list(range(0, ZT - zb_rows, zb_rows)) + [ZT - zb_rows]
    for off in offs:
        pltpu.sync_copy(zbuf.at[pl.ds(0, zb_rows)],
                        acc_sh.at[pl.ds(s * ZT + off, zb_rows)])


def _sc_degree(colm, cpad):
    @functools.partial(
        pl.kernel,
        out_type=jax.ShapeDtypeStruct((NC, N, H), F32),
        mesh=_mesh(),
        scratch_types=[
            pltpu.VMEM((CH, IR), I32),
            pltpu.VMEM((KE, H), F32),
            pltpu.VMEM_SHARED((NA, H), F32),
        ],
        compiler_params=_sc_params(),
    )
    def k(col_hbm, cpad_hbm, deg_hbm, ci, ones_v, acc_sh):
        c = lax.axis_index("c")
        s = lax.axis_index("s")
        _zero_acc(ones_v, ZB, acc_sh, s)
        plsc.subcore_barrier()

        @pl.loop(0, KE)
        def _(j):
            ones_v[j, :] = jnp.ones((H,), F32)

        rows_w = DR // (NC * NS)
        base = (c * NS + s) * rows_w
        HCH = CH // 2

        @pl.loop(0, rows_w, step=CH)
        def _(r):
            g = base + r

            @pl.when(g + CH <= ER)
            def _():
                pltpu.sync_copy(col_hbm.at[pl.ds(g, CH)], ci)

            @pl.when(g >= ER)
            def _():
                pltpu.sync_copy(cpad_hbm.at[pl.ds(g - ER, CH)], ci)

            @pl.when(jnp.logical_and(g < ER, g + CH > ER))
            def _():
                pltpu.sync_copy(col_hbm.at[pl.ds(g, HCH)],
                                ci.at[pl.ds(0, HCH)])
                pltpu.sync_copy(cpad_hbm.at[pl.ds(0, HCH)],
                                ci.at[pl.ds(HCH, HCH)])

            for i in range(CH):
                pltpu.sync_copy(ones_v.at[pl.ds(0, IR)],
                                acc_sh.at[ci.at[i]], add=True)

        plsc.subcore_barrier()
        pltpu.sync_copy(acc_sh.at[pl.ds(s * OT, OT)],
                        deg_hbm.at[c].at[pl.ds(s * OT, OT)])

    return k(colm, cpad)


def _sc_layer(rowm, rpad, colm, cpad, yflat):
    @functools.partial(
        pl.kernel,
        out_type=jax.ShapeDtypeStruct((NC * N, H), F32),
        mesh=_mesh(),
        scratch_types=[
            pltpu.VMEM((LCH, IR), I32),
            pltpu.VMEM((LCH, IR), I32),
            pltpu.VMEM((LCH, IR), I32),
            pltpu.VMEM((LCH, IR), I32),
            pltpu.VMEM((LKE, H), F32),
            pltpu.VMEM((LKE, H), F32),
            pltpu.VMEM_SHARED((NA, H), F32),
            pltpu.SemaphoreType.DMA,
            pltpu.SemaphoreType.DMA,
            pltpu.SemaphoreType.DMA,
        ],
        compiler_params=_sc_params(),
    )
    def k(row_hbm, rpad_hbm, col_hbm, cpad_hbm, y_hbm, s_hbm,
          ri0, ci0, ri1, ci1, rv0, rv1, acc_sh, sem_i, sem_g, sem_s):
        c = lax.axis_index("c")
        s = lax.axis_index("s")
        _zero_acc(rv0, LKE, acc_sh, s)
        plsc.subcore_barrier()

        rows_t = DR // NS
        base = s * rows_t
        off0 = c * N

        def drain_scatters(ci_, rv_):
            for i in range(LCH):
                pltpu.make_async_copy(rv_.at[pl.ds(i * IR, IR)],
                                      acc_sh.at[ci_.at[i]], sem_s).wait()

        def chunk(r, ri_, ci_, rv_):
            @pl.when(r >= 2 * LCH)
            def _():
                drain_scatters(ci_, rv_)

            g = base + r

            @pl.when(g + LCH <= ER)
            def _():
                pltpu.async_copy(row_hbm.at[pl.ds(g, LCH)], ri_, sem_i)
                pltpu.async_copy(col_hbm.at[pl.ds(g, LCH)], ci_, sem_i)
                pltpu.make_async_copy(row_hbm.at[pl.ds(g, LCH)], ri_,
                                      sem_i).wait()
                pltpu.make_async_copy(col_hbm.at[pl.ds(g, LCH)], ci_,
                                      sem_i).wait()

            @pl.when(g + LCH > ER)
            def _():
                pltpu.async_copy(rpad_hbm.at[pl.ds(g - ER, LCH)], ri_, sem_i)
                pltpu.async_copy(cpad_hbm.at[pl.ds(g - ER, LCH)], ci_, sem_i)
                pltpu.make_async_copy(rpad_hbm.at[pl.ds(g - ER, LCH)], ri_,
                                      sem_i).wait()
                pltpu.make_async_copy(cpad_hbm.at[pl.ds(g - ER, LCH)], ci_,
                                      sem_i).wait()

            @pl.loop(0, LCH)
            def _(i):
                @pl.loop(0, IR, step=16)
                def _(l):
                    ri_[i, pl.ds(l, 16)] = ri_[i, pl.ds(l, 16)] + off0

            for i in range(LCH):
                pltpu.async_copy(y_hbm.at[ri_.at[i]],
                                 rv_.at[pl.ds(i * IR, IR)], sem_g)
            for i in range(LCH):
                pltpu.make_async_copy(y_hbm.at[ri_.at[i]],
                                      rv_.at[pl.ds(i * IR, IR)],
                                      sem_g).wait()
            for i in range(LCH):
                pltpu.async_copy(rv_.at[pl.ds(i * IR, IR)],
                                 acc_sh.at[ci_.at[i]], sem_s, add=True)

        @pl.loop(0, rows_t, step=2 * LCH)
        def _(r):
            chunk(r, ri0, ci0, rv0)
            chunk(r + LCH, ri1, ci1, rv1)

        drain_scatters(ci0, rv0)
        drain_scatters(ci1, rv1)
        plsc.subcore_barrier()
        pltpu.sync_copy(acc_sh.at[pl.ds(s * OT, OT)],
                        s_hbm.at[pl.ds(c * N + s * OT, OT)])

    return k(rowm, rpad, colm, cpad, yflat)


SCH = 4
SKE = SCH * IR


def _sc_score(rowm, colm, pad, tbl):
    @functools.partial(
        pl.kernel,
        out_type=jax.ShapeDtypeStruct((EP,), F32),
        mesh=_mesh(),
        scratch_types=[
            pltpu.VMEM((SCH, IR), I32),
            pltpu.VMEM((SCH, IR), I32),
            pltpu.VMEM((SCH, IR), I32),
            pltpu.VMEM((SCH, IR), I32),
            pltpu.VMEM((SKE, D), F32),
            pltpu.VMEM((SKE, D), F32),
            pltpu.VMEM((SKE, D), F32),
            pltpu.VMEM((SKE, D), F32),
            pltpu.VMEM((SKE,), F32),
            pltpu.VMEM((SKE,), F32),
            pltpu.SemaphoreType.DMA,
            pltpu.SemaphoreType.DMA,
            pltpu.SemaphoreType.DMA,
        ],
        compiler_params=_sc_params(layout_passes=False),
    )
    def k(row_hbm, col_hbm, pad_hbm, tbl_hbm, out_hbm, ri0, ci0, ri1, ci1,
          av0, bv0, av1, bv1, sv0, sv1, sem_i, sem_g, sem_o):
        c = lax.axis_index("c")
        s = lax.axis_index("s")
        lane = lax.iota(I32, 16)
        rows_w = DR // (NC * NS)
        base = (s * NC + c) * rows_w

        def fire(r, ri_, ci_, av_, bv_):
            g = base + r

            @pl.when(g + SCH <= ER)
            def _():
                pltpu.async_copy(row_hbm.at[pl.ds(g, SCH)], ri_, sem_i)
                pltpu.async_copy(col_hbm.at[pl.ds(g, SCH)], ci_, sem_i)
                pltpu.make_async_copy(row_hbm.at[pl.ds(g, SCH)], ri_,
                                      sem_i).wait()
                pltpu.make_async_copy(col_hbm.at[pl.ds(g, SCH)], ci_,
                                      sem_i).wait()

            @pl.when(g + SCH > ER)
            def _():
                pltpu.async_copy(pad_hbm.at[pl.ds(g - ER, SCH)], ri_, sem_i)
                pltpu.async_copy(pad_hbm.at[pl.ds(g - ER, SCH)], ci_, sem_i)
                pltpu.make_async_copy(pad_hbm.at[pl.ds(g - ER, SCH)], ri_,
                                      sem_i).wait()
                pltpu.make_async_copy(pad_hbm.at[pl.ds(g - ER, SCH)], ci_,
                                      sem_i).wait()
            for i in range(SCH):
                pltpu.async_copy(tbl_hbm.at[ri_.at[i]],
                                 av_.at[pl.ds(i * IR, IR)], sem_g)
                pltpu.async_copy(tbl_hbm.at[ci_.at[i]],
                                 bv_.at[pl.ds(i * IR, IR)], sem_g)

        def process(r, ri_, ci_, av_, bv_, sv_):
            for i in range(SCH):
                pltpu.make_async_copy(tbl_hbm.at[ri_.at[i]],
                                      av_.at[pl.ds(i * IR, IR)], sem_g).wait()
                pltpu.make_async_copy(tbl_hbm.at[ci_.at[i]],
                                      bv_.at[pl.ds(i * IR, IR)], sem_g).wait()

            @pl.when(r >= 2 * SCH)
            def _():
                pltpu.make_async_copy(
                    sv_, out_hbm.at[pl.ds((base + r) * IR, SKE)], sem_o).wait()

            @pl.loop(0, SKE, step=16)
            def _(j0):
                ridx = lane + j0
                acc = jnp.zeros((16,), F32)
                for d in range(D):
                    cidx = (lane + d) & (D - 1)
                    ga = plsc.load_gather(av_, [ridx, cidx])
                    gb = plsc.load_gather(bv_, [ridx, cidx])
                    acc = acc + ga * gb
                sv_[pl.ds(j0, 16)] = acc

            pltpu.async_copy(sv_, out_hbm.at[pl.ds((base + r) * IR, SKE)],
                             sem_o)

        fire(0, ri0, ci0, av0, bv0)

        @pl.loop(0, rows_w, step=2 * SCH)
        def _(r):
            fire(r + SCH, ri1, ci1, av1, bv1)
            process(r, ri0, ci0, av0, bv0, sv0)

            @pl.when(r + 2 * SCH < rows_w)
            def _():
                fire(r + 2 * SCH, ri0, ci0, av0, bv0)

            process(r + SCH, ri1, ci1, av1, bv1, sv1)

        pltpu.make_async_copy(sv0, out_hbm.at[pl.ds(base * IR, SKE)],
                              sem_o).wait()
        pltpu.make_async_copy(sv1, out_hbm.at[pl.ds(base * IR, SKE)],
                              sem_o).wait()

    return k(rowm, colm, pad, tbl)


WR = N // (NC * NS)


def _offs(total, chb):
    return list(range(0, total - chb, chb)) + [total - chb]


def _sc_prep(degp, emb):
    CHB = 1024

    @functools.partial(
        pl.kernel,
        out_type=[
            jax.ShapeDtypeStruct((N, H), F32),
            jax.ShapeDtypeStruct((NC * N, H), F32),
        ],
        mesh=_mesh(),
        scratch_types=[
            pltpu.VMEM((CHB, H), F32),
            pltpu.VMEM((CHB, H), F32),
            pltpu.VMEM((CHB, D), F32),
            pltpu.VMEM((CHB, H), F32),
            pltpu.VMEM((CHB, H), F32),
            pltpu.VMEM((CHB, H), F32),
        ],
        compiler_params=_sc_params(),
    )
    def k(degp_hbm, emb_hbm, dis_hbm, y_hbm, d0v, d1v, ev, dv, y0v, y1v):
        c = lax.axis_index("c")
        s = lax.axis_index("s")
        w0 = (c * NS + s) * WR

        for off in _offs(WR, CHB):
            row0 = w0 + off
            pltpu.sync_copy(degp_hbm.at[0].at[pl.ds(row0, CHB)], d0v)
            pltpu.sync_copy(degp_hbm.at[1].at[pl.ds(row0, CHB)], d1v)
            pltpu.sync_copy(emb_hbm.at[pl.ds(row0, CHB)], ev)

            @pl.loop(0, CHB)
            def _(j):
                deg = d0v[j, :] + d1v[j, :]
                xi = lax.bitcast_convert_type(deg, I32)
                yi = jnp.int32(0x5F3759DF) - lax.shift_right_logical(
                    xi, jnp.ones_like(xi))
                y = lax.bitcast_convert_type(yi, F32)
                h = deg * jnp.float32(0.5)
                for _ in range(4):
                    y = y * (jnp.float32(1.5) - h * y * y)
                dis = jnp.where(deg > 0.0, y, 0.0)
                dv[j, :] = dis
                y0v[j, :] = dis * ev[j, pl.ds(0, H)]
                y1v[j, :] = dis * ev[j, pl.ds(H, H)]

            pltpu.sync_copy(dv, dis_hbm.at[pl.ds(row0, CHB)])
            pltpu.sync_copy(y0v, y_hbm.at[pl.ds(row0, CHB)])
            pltpu.sync_copy(y1v, y_hbm.at[pl.ds(N + row0, CHB)])

    return k(degp, emb)


def _sc_mid(s1, dis16):
    CHB = 1024

    @functools.partial(
        pl.kernel,
        out_type=[
            jax.ShapeDtypeStruct((NC * N, H), F32),
            jax.ShapeDtypeStruct((NC * N, H), F32),
        ],
        mesh=_mesh(),
        scratch_types=[
            pltpu.VMEM((CHB, H), F32),
            pltpu.VMEM((CHB, H), F32),
            pltpu.VMEM((CHB, H), F32),
            pltpu.VMEM((CHB, H), F32),
            pltpu.VMEM((CHB, H), F32),
            pltpu.VMEM((CHB, H), F32),
            pltpu.VMEM((CHB, H), F32),
        ],
        compiler_params=_sc_params(),
    )
    def k(s_hbm, dis_hbm, x_hbm, y_hbm, s0v, s1v, dv, x0v, x1v, y0v, y1v):
        c = lax.axis_index("c")
        s = lax.axis_index("s")
        w0 = (c * NS + s) * WR

        for off in _offs(WR, CHB):
            row0 = w0 + off
            pltpu.sync_copy(s_hbm.at[pl.ds(row0, CHB)], s0v)
            pltpu.sync_copy(s_hbm.at[pl.ds(N + row0, CHB)], s1v)
            pltpu.sync_copy(dis_hbm.at[pl.ds(row0, CHB)], dv)

            @pl.loop(0, CHB)
            def _(j):
                dis = dv[j, :]
                xa = dis * s0v[j, :]
                xb = dis * s1v[j, :]
                x0v[j, :] = xa
                x1v[j, :] = xb
                y0v[j, :] = dis * xa
                y1v[j, :] = dis * xb

            pltpu.sync_copy(x0v, x_hbm.at[pl.ds(row0, CHB)])
            pltpu.sync_copy(x1v, x_hbm.at[pl.ds(N + row0, CHB)])
            pltpu.sync_copy(y0v, y_hbm.at[pl.ds(row0, CHB)])
            pltpu.sync_copy(y1v, y_hbm.at[pl.ds(N + row0, CHB)])

    return k(s1, dis16)


def _sc_final(emb, x1, s2, dis16):
    CHB = 768

    @functools.partial(
        pl.kernel,
        out_type=jax.ShapeDtypeStruct((N, D), F32),
        mesh=_mesh(),
        scratch_types=[
            pltpu.VMEM((CHB, D), F32),
            pltpu.VMEM((CHB, H), F32),
            pltpu.VMEM((CHB, H), F32),
            pltpu.VMEM((CHB, H), F32),
            pltpu.VMEM((CHB, H), F32),
            pltpu.VMEM((CHB, H), F32),
            pltpu.VMEM((CHB, D), F32),
        ],
        compiler_params=_sc_params(),
    )
    def k(emb_hbm, x_hbm, s_hbm, dis_hbm, tbl_hbm,
          ev, x0v, x1v, s0v, s1v, dv, ov):
        c = lax.axis_index("c")
        s = lax.axis_index("s")
        w0 = (c * NS + s) * WR
        athird = jnp.float32(1.0 / 3.0)

        for off in _offs(WR, CHB):
            row0 = w0 + off
            pltpu.sync_copy(emb_hbm.at[pl.ds(row0, CHB)], ev)
            pltpu.sync_copy(x_hbm.at[pl.ds(row0, CHB)], x0v)
            pltpu.sync_copy(x_hbm.at[pl.ds(N + row0, CHB)], x1v)
            pltpu.sync_copy(s_hbm.at[pl.ds(row0, CHB)], s0v)
            pltpu.sync_copy(s_hbm.at[pl.ds(N + row0, CHB)], s1v)
            pltpu.sync_copy(dis_hbm.at[pl.ds(row0, CHB)], dv)

            @pl.loop(0, CHB)
            def _(j):
                dis = dv[j, :]
                ov[j, pl.ds(0, H)] = (ev[j, pl.ds(0, H)] + x0v[j, :]
                                      + dis * s0v[j, :]) * athird
                ov[j, pl.ds(H, H)] = (ev[j, pl.ds(H, H)] + x1v[j, :]
                                      + dis * s1v[j, :]) * athird

            pltpu.sync_copy(ov, tbl_hbm.at[pl.ds(row0, CHB)])

    return k(emb, x1, s2, dis16)


def kernel(edge_index, emb):
    rowm = edge_index[0].reshape(ER, IR)
    colm = edge_index[1].reshape(ER, IR)
    pad = jnp.arange(PAD, dtype=I32)
    padg = (pad % N).reshape(PR, IR)
    pads = (N + (pad % NSINK)).reshape(PR, IR)

    degp = _sc_degree(colm, pads)
    dis16, y1 = _sc_prep(degp, emb)
    s1 = _sc_layer(rowm, padg, colm, pads, y1)
    x1, y2 = _sc_mid(s1, dis16)
    s2 = _sc_layer(rowm, padg, colm, pads, y2)
    out_tbl = _sc_final(emb, x1, s2, dis16)
    return _sc_score(rowm, colm, padg, out_tbl)[:E]


# speedup vs baseline: 1.1894x; 1.0191x over previous
import dataclasses
import functools

import jax
import jax.numpy as jnp
from jax import lax
from jax.experimental import pallas as pl
from jax.experimental.pallas import tpu as pltpu
from jax.experimental.pallas import tpu_sc as plsc

N = 100000
E = 1600000
EP = 1638400
PAD = EP - E
D = 32
H = 16
NC = 2
NS = 16
NSINK = 128
NA = N + NSINK
IR = 128
CH = 8
KE = CH * IR
LCH = 4
LKE = LCH * IR
DR = EP // IR
ER = E // IR
PR = PAD // IR
ZB = 1024
ZT = NA // NS
OT = N // NS
F32 = jnp.float32
I32 = jnp.int32


def _mesh():
    return plsc.VectorSubcoreMesh(core_axis_name="c", subcore_axis_name="s")


def _sc_params(layout_passes=True):
    cp = pltpu.CompilerParams()
    fields = getattr(pltpu.CompilerParams, "__dataclass_fields__", {})
    if "use_tc_tiling_on_sc" in fields:
        cp = dataclasses.replace(cp, use_tc_tiling_on_sc=False)
    if not layout_passes and "needs_layout_passes" in fields:
        cp = dataclasses.replace(cp, needs_layout_passes=False)
    return cp


def _zero_acc(zbuf, zb_rows, acc_sh, s):
    @pl.loop(0, zb_rows)
    def _(j):
        zbuf[j, :] = jnp.zeros((H,), F32)

    offs = list(range(0, ZT - zb_rows, zb_rows)) + [ZT - zb_rows]
    for off in offs:
        pltpu.sync_copy(zbuf.at[pl.ds(0, zb_rows)],
                        acc_sh.at[pl.ds(s * ZT + off, zb_rows)])


def _sc_degree(colm, cpad):
    @functools.partial(
        pl.kernel,
        out_type=jax.ShapeDtypeStruct((NC, N, H), F32),
        mesh=_mesh(),
        scratch_types=[
            pltpu.VMEM((CH, IR), I32),
            pltpu.VMEM((KE, H), F32),
            pltpu.VMEM_SHARED((NA, H), F32),
        ],
        compiler_params=_sc_params(),
    )
    def k(col_hbm, cpad_hbm, deg_hbm, ci, ones_v, acc_sh):
        c = lax.axis_index("c")
        s = lax.axis_index("s")
        _zero_acc(ones_v, ZB, acc_sh, s)
        plsc.subcore_barrier()

        @pl.loop(0, KE)
        def _(j):
            ones_v[j, :] = jnp.ones((H,), F32)

        rows_w = DR // (NC * NS)
        base = (c * NS + s) * rows_w
        HCH = CH // 2

        @pl.loop(0, rows_w, step=CH)
        def _(r):
            g = base + r

            @pl.when(g + CH <= ER)
            def _():
                pltpu.sync_copy(col_hbm.at[pl.ds(g, CH)], ci)

            @pl.when(g >= ER)
            def _():
                pltpu.sync_copy(cpad_hbm.at[pl.ds(g - ER, CH)], ci)

            @pl.when(jnp.logical_and(g < ER, g + CH > ER))
            def _():
                pltpu.sync_copy(col_hbm.at[pl.ds(g, HCH)],
                                ci.at[pl.ds(0, HCH)])
                pltpu.sync_copy(cpad_hbm.at[pl.ds(0, HCH)],
                                ci.at[pl.ds(HCH, HCH)])

            for i in range(CH):
                pltpu.sync_copy(ones_v.at[pl.ds(0, IR)],
                                acc_sh.at[ci.at[i]], add=True)

        plsc.subcore_barrier()
        pltpu.sync_copy(acc_sh.at[pl.ds(s * OT, OT)],
                        deg_hbm.at[c].at[pl.ds(s * OT, OT)])

    return k(colm, cpad)


def _sc_layer(rowm, rpad, colm, cpad, yflat):
    @functools.partial(
        pl.kernel,
        out_type=jax.ShapeDtypeStruct((NC * N, H), F32),
        mesh=_mesh(),
        scratch_types=[
            pltpu.VMEM((LCH, IR), I32),
            pltpu.VMEM((LCH, IR), I32),
            pltpu.VMEM((LCH, IR), I32),
            pltpu.VMEM((LCH, IR), I32),
            pltpu.VMEM((LKE, H), F32),
            pltpu.VMEM((LKE, H), F32),
            pltpu.VMEM_SHARED((NA, H), F32),
            pltpu.SemaphoreType.DMA,
            pltpu.SemaphoreType.DMA,
            pltpu.SemaphoreType.DMA,
        ],
        compiler_params=_sc_params(),
    )
    def k(row_hbm, rpad_hbm, col_hbm, cpad_hbm, y_hbm, s_hbm,
          ri0, ci0, ri1, ci1, rv0, rv1, acc_sh, sem_i, sem_g, sem_s):
        c = lax.axis_index("c")
        s = lax.axis_index("s")
        _zero_acc(rv0, LKE, acc_sh, s)
        plsc.subcore_barrier()

        rows_t = DR // NS
        base = s * rows_t
        off0 = c * N

        def drain_scatters(ci_, rv_):
            for i in range(LCH):
                pltpu.make_async_copy(rv_.at[pl.ds(i * IR, IR)],
                                      acc_sh.at[ci_.at[i]], sem_s).wait()

        def chunk(r, ri_, ci_, rv_):
            @pl.when(r >= 2 * LCH)
            def _():
                drain_scatters(ci_, rv_)

            g = base + r

            @pl.when(g + LCH <= ER)
            def _():
                pltpu.async_copy(row_hbm.at[pl.ds(g, LCH)], ri_, sem_i)
                pltpu.async_copy(col_hbm.at[pl.ds(g, LCH)], ci_, sem_i)
                pltpu.make_async_copy(row_hbm.at[pl.ds(g, LCH)], ri_,
                                      sem_i).wait()
                pltpu.make_async_copy(col_hbm.at[pl.ds(g, LCH)], ci_,
                                      sem_i).wait()

            @pl.when(g + LCH > ER)
            def _():
                pltpu.async_copy(rpad_hbm.at[pl.ds(g - ER, LCH)], ri_, sem_i)
                pltpu.async_copy(cpad_hbm.at[pl.ds(g - ER, LCH)], ci_, sem_i)
                pltpu.make_async_copy(rpad_hbm.at[pl.ds(g - ER, LCH)], ri_,
                                      sem_i).wait()
                pltpu.make_async_copy(cpad_hbm.at[pl.ds(g - ER, LCH)], ci_,
                                      sem_i).wait()

            @pl.loop(0, LCH)
            def _(i):
                @pl.loop(0, IR, step=16)
                def _(l):
                    ri_[i, pl.ds(l, 16)] = ri_[i, pl.ds(l, 16)] + off0

            for i in range(LCH):
                pltpu.async_copy(y_hbm.at[ri_.at[i]],
                                 rv_.at[pl.ds(i * IR, IR)], sem_g)
            for i in range(LCH):
                pltpu.make_async_copy(y_hbm.at[ri_.at[i]],
                                      rv_.at[pl.ds(i * IR, IR)],
                                      sem_g).wait()
            for i in range(LCH):
                pltpu.async_copy(rv_.at[pl.ds(i * IR, IR)],
                                 acc_sh.at[ci_.at[i]], sem_s, add=True)

        @pl.loop(0, rows_t, step=2 * LCH)
        def _(r):
            chunk(r, ri0, ci0, rv0)
            chunk(r + LCH, ri1, ci1, rv1)

        drain_scatters(ci0, rv0)
        drain_scatters(ci1, rv1)
        plsc.subcore_barrier()
        pltpu.sync_copy(acc_sh.at[pl.ds(s * OT, OT)],
                        s_hbm.at[pl.ds(c * N + s * OT, OT)])

    return k(rowm, rpad, colm, cpad, yflat)


SCH = 4
SKE = SCH * IR


def _sc_score(rowm, colm, pad, tbl):
    @functools.partial(
        pl.kernel,
        out_type=jax.ShapeDtypeStruct((EP,), F32),
        mesh=_mesh(),
        scratch_types=[
            pltpu.VMEM((SCH, IR), I32),
            pltpu.VMEM((SCH, IR), I32),
            pltpu.VMEM((SCH, IR), I32),
            pltpu.VMEM((SCH, IR), I32),
            pltpu.VMEM((SKE, D), F32),
            pltpu.VMEM((SKE, D), F32),
            pltpu.VMEM((SKE, D), F32),
            pltpu.VMEM((SKE, D), F32),
            pltpu.VMEM((SKE,), F32),
            pltpu.VMEM((SKE,), F32),
            pltpu.SemaphoreType.DMA,
            pltpu.SemaphoreType.DMA,
            pltpu.SemaphoreType.DMA,
        ],
        compiler_params=_sc_params(layout_passes=False),
    )
    def k(row_hbm, col_hbm, pad_hbm, tbl_hbm, out_hbm, ri0, ci0, ri1, ci1,
          av0, bv0, av1, bv1, sv0, sv1, sem_i, sem_g, sem_o):
        c = lax.axis_index("c")
        s = lax.axis_index("s")
        lane = lax.iota(I32, 16)
        rows_w = DR // (NC * NS)
        base = (s * NC + c) * rows_w

        def fire(r, ri_, ci_, av_, bv_):
            g = base + r

            @pl.when(g + SCH <= ER)
            def _():
                pltpu.async_copy(row_hbm.at[pl.ds(g, SCH)], ri_, sem_i)
                pltpu.async_copy(col_hbm.at[pl.ds(g, SCH)], ci_, sem_i)
                pltpu.make_async_copy(row_hbm.at[pl.ds(g, SCH)], ri_,
                                      sem_i).wait()
                pltpu.make_async_copy(col_hbm.at[pl.ds(g, SCH)], ci_,
                                      sem_i).wait()

            @pl.when(g + SCH > ER)
            def _():
                pltpu.async_copy(pad_hbm.at[pl.ds(g - ER, SCH)], ri_, sem_i)
                pltpu.async_copy(pad_hbm.at[pl.ds(g - ER, SCH)], ci_, sem_i)
                pltpu.make_async_copy(pad_hbm.at[pl.ds(g - ER, SCH)], ri_,
                                      sem_i).wait()
                pltpu.make_async_copy(pad_hbm.at[pl.ds(g - ER, SCH)], ci_,
                                      sem_i).wait()
            for i in range(SCH):
                pltpu.async_copy(tbl_hbm.at[ri_.at[i]],
                                 av_.at[pl.ds(i * IR, IR)], sem_g)
                pltpu.async_copy(tbl_hbm.at[ci_.at[i]],
                                 bv_.at[pl.ds(i * IR, IR)], sem_g)

        def process(r, ri_, ci_, av_, bv_, sv_):
            for i in range(SCH):
                pltpu.make_async_copy(tbl_hbm.at[ri_.at[i]],
                                      av_.at[pl.ds(i * IR, IR)], sem_g).wait()
                pltpu.make_async_copy(tbl_hbm.at[ci_.at[i]],
                                      bv_.at[pl.ds(i * IR, IR)], sem_g).wait()

            @pl.when(r >= 2 * SCH)
            def _():
                pltpu.make_async_copy(
                    sv_, out_hbm.at[pl.ds((base + r) * IR, SKE)], sem_o).wait()

            @pl.loop(0, SKE, step=16)
            def _(j0):
                ridx = lane + j0
                acc = jnp.zeros((16,), F32)
                for d in range(D):
                    cidx = (lane + d) & (D - 1)
                    ga = plsc.load_gather(av_, [ridx, cidx])
                    gb = plsc.load_gather(bv_, [ridx, cidx])
                    acc = acc + ga * gb
                sv_[pl.ds(j0, 16)] = acc

            pltpu.async_copy(sv_, out_hbm.at[pl.ds((base + r) * IR, SKE)],
                             sem_o)

        fire(0, ri0, ci0, av0, bv0)

        @pl.loop(0, rows_w, step=2 * SCH)
        def _(r):
            fire(r + SCH, ri1, ci1, av1, bv1)
            process(r, ri0, ci0, av0, bv0, sv0)

            @pl.when(r + 2 * SCH < rows_w)
            def _():
                fire(r + 2 * SCH, ri0, ci0, av0, bv0)

            process(r + SCH, ri1, ci1, av1, bv1, sv1)

        pltpu.make_async_copy(sv0, out_hbm.at[pl.ds(base * IR, SKE)],
                              sem_o).wait()
        pltpu.make_async_copy(sv1, out_hbm.at[pl.ds(base * IR, SKE)],
                              sem_o).wait()

    return k(rowm, colm, pad, tbl)


WR = N // (NC * NS)


def _offs(total, chb):
    return list(range(0, total - chb, chb)) + [total - chb]


def _sc_prep(degp, emb):
    CHB = 1024

    @functools.partial(
        pl.kernel,
        out_type=[
            jax.ShapeDtypeStruct((N, H), F32),
            jax.ShapeDtypeStruct((NC * N, H), F32),
        ],
        mesh=_mesh(),
        scratch_types=[
            pltpu.VMEM((CHB, H), F32),
            pltpu.VMEM((CHB, H), F32),
            pltpu.VMEM((CHB, D), F32),
            pltpu.VMEM((CHB, H), F32),
            pltpu.VMEM((CHB, H), F32),
            pltpu.VMEM((CHB, H), F32),
            pltpu.SemaphoreType.DMA,
        ],
        compiler_params=_sc_params(),
    )
    def k(degp_hbm, emb_hbm, dis_hbm, y_hbm, d0v, d1v, ev, dv, y0v, y1v, sem):
        c = lax.axis_index("c")
        s = lax.axis_index("s")
        w0 = (c * NS + s) * WR

        for off in _offs(WR, CHB):
            row0 = w0 + off
            pltpu.async_copy(degp_hbm.at[0].at[pl.ds(row0, CHB)], d0v, sem)
            pltpu.async_copy(degp_hbm.at[1].at[pl.ds(row0, CHB)], d1v, sem)
            pltpu.async_copy(emb_hbm.at[pl.ds(row0, CHB)], ev, sem)
            pltpu.make_async_copy(degp_hbm.at[0].at[pl.ds(row0, CHB)],
                                  d0v, sem).wait()
            pltpu.make_async_copy(degp_hbm.at[1].at[pl.ds(row0, CHB)],
                                  d1v, sem).wait()
            pltpu.make_async_copy(emb_hbm.at[pl.ds(row0, CHB)],
                                  ev, sem).wait()

            @pl.loop(0, CHB)
            def _(j):
                deg = d0v[j, :] + d1v[j, :]
                xi = lax.bitcast_convert_type(deg, I32)
                yi = jnp.int32(0x5F3759DF) - lax.shift_right_logical(
                    xi, jnp.ones_like(xi))
                y = lax.bitcast_convert_type(yi, F32)
                h = deg * jnp.float32(0.5)
                for _ in range(4):
                    y = y * (jnp.float32(1.5) - h * y * y)
                dis = jnp.where(deg > 0.0, y, 0.0)
                dv[j, :] = dis
                y0v[j, :] = dis * ev[j, pl.ds(0, H)]
                y1v[j, :] = dis * ev[j, pl.ds(H, H)]

            pltpu.async_copy(dv, dis_hbm.at[pl.ds(row0, CHB)], sem)
            pltpu.async_copy(y0v, y_hbm.at[pl.ds(row0, CHB)], sem)
            pltpu.async_copy(y1v, y_hbm.at[pl.ds(N + row0, CHB)], sem)
            pltpu.make_async_copy(dv, dis_hbm.at[pl.ds(row0, CHB)],
                                  sem).wait()
            pltpu.make_async_copy(y0v, y_hbm.at[pl.ds(row0, CHB)],
                                  sem).wait()
            pltpu.make_async_copy(y1v, y_hbm.at[pl.ds(N + row0, CHB)],
                                  sem).wait()

    return k(degp, emb)


def _sc_mid(s1, dis16):
    CHB = 1024

    @functools.partial(
        pl.kernel,
        out_type=[
            jax.ShapeDtypeStruct((NC * N, H), F32),
            jax.ShapeDtypeStruct((NC * N, H), F32),
        ],
        mesh=_mesh(),
        scratch_types=[
            pltpu.VMEM((CHB, H), F32),
            pltpu.VMEM((CHB, H), F32),
            pltpu.VMEM((CHB, H), F32),
            pltpu.VMEM((CHB, H), F32),
            pltpu.VMEM((CHB, H), F32),
            pltpu.VMEM((CHB, H), F32),
            pltpu.VMEM((CHB, H), F32),
            pltpu.SemaphoreType.DMA,
        ],
        compiler_params=_sc_params(),
    )
    def k(s_hbm, dis_hbm, x_hbm, y_hbm, s0v, s1v, dv, x0v, x1v, y0v, y1v,
          sem):
        c = lax.axis_index("c")
        s = lax.axis_index("s")
        w0 = (c * NS + s) * WR

        for off in _offs(WR, CHB):
            row0 = w0 + off
            pltpu.async_copy(s_hbm.at[pl.ds(row0, CHB)], s0v, sem)
            pltpu.async_copy(s_hbm.at[pl.ds(N + row0, CHB)], s1v, sem)
            pltpu.async_copy(dis_hbm.at[pl.ds(row0, CHB)], dv, sem)
            pltpu.make_async_copy(s_hbm.at[pl.ds(row0, CHB)],
                                  s0v, sem).wait()
            pltpu.make_async_copy(s_hbm.at[pl.ds(N + row0, CHB)],
                                  s1v, sem).wait()
            pltpu.make_async_copy(dis_hbm.at[pl.ds(row0, CHB)],
                                  dv, sem).wait()

            @pl.loop(0, CHB)
            def _(j):
                dis = dv[j, :]
                xa = dis * s0v[j, :]
                xb = dis * s1v[j, :]
                x0v[j, :] = xa
                x1v[j, :] = xb
                y0v[j, :] = dis * xa
                y1v[j, :] = dis * xb

            pltpu.async_copy(x0v, x_hbm.at[pl.ds(row0, CHB)], sem)
            pltpu.async_copy(x1v, x_hbm.at[pl.ds(N + row0, CHB)], sem)
            pltpu.async_copy(y0v, y_hbm.at[pl.ds(row0, CHB)], sem)
            pltpu.async_copy(y1v, y_hbm.at[pl.ds(N + row0, CHB)], sem)
            pltpu.make_async_copy(x0v, x_hbm.at[pl.ds(row0, CHB)],
                                  sem).wait()
            pltpu.make_async_copy(x1v, x_hbm.at[pl.ds(N + row0, CHB)],
                                  sem).wait()
            pltpu.make_async_copy(y0v, y_hbm.at[pl.ds(row0, CHB)],
                                  sem).wait()
            pltpu.make_async_copy(y1v, y_hbm.at[pl.ds(N + row0, CHB)],
                                  sem).wait()

    return k(s1, dis16)


def _sc_final(emb, x1, s2, dis16):
    CHB = 768

    @functools.partial(
        pl.kernel,
        out_type=jax.ShapeDtypeStruct((N, D), F32),
        mesh=_mesh(),
        scratch_types=[
            pltpu.VMEM((CHB, D), F32),
            pltpu.VMEM((CHB, H), F32),
            pltpu.VMEM((CHB, H), F32),
            pltpu.VMEM((CHB, H), F32),
            pltpu.VMEM((CHB, H), F32),
            pltpu.VMEM((CHB, H), F32),
            pltpu.VMEM((CHB, D), F32),
            pltpu.SemaphoreType.DMA,
        ],
        compiler_params=_sc_params(),
    )
    def k(emb_hbm, x_hbm, s_hbm, dis_hbm, tbl_hbm,
          ev, x0v, x1v, s0v, s1v, dv, ov, sem):
        c = lax.axis_index("c")
        s = lax.axis_index("s")
        w0 = (c * NS + s) * WR
        athird = jnp.float32(1.0 / 3.0)

        for off in _offs(WR, CHB):
            row0 = w0 + off
            reads = [
                (emb_hbm.at[pl.ds(row0, CHB)], ev),
                (x_hbm.at[pl.ds(row0, CHB)], x0v),
                (x_hbm.at[pl.ds(N + row0, CHB)], x1v),
                (s_hbm.at[pl.ds(row0, CHB)], s0v),
                (s_hbm.at[pl.ds(N + row0, CHB)], s1v),
                (dis_hbm.at[pl.ds(row0, CHB)], dv),
            ]
            for src, dst in reads:
                pltpu.async_copy(src, dst, sem)
            for src, dst in reads:
                pltpu.make_async_copy(src, dst, sem).wait()

            @pl.loop(0, CHB)
            def _(j):
                dis = dv[j, :]
                ov[j, pl.ds(0, H)] = (ev[j, pl.ds(0, H)] + x0v[j, :]
                                      + dis * s0v[j, :]) * athird
                ov[j, pl.ds(H, H)] = (ev[j, pl.ds(H, H)] + x1v[j, :]
                                      + dis * s1v[j, :]) * athird

            pltpu.sync_copy(ov, tbl_hbm.at[pl.ds(row0, CHB)])

    return k(emb, x1, s2, dis16)


def kernel(edge_index, emb):
    rowm = edge_index[0].reshape(ER, IR)
    colm = edge_index[1].reshape(ER, IR)
    pad = jnp.arange(PAD, dtype=I32)
    padg = (pad % N).reshape(PR, IR)
    pads = (N + (pad % NSINK)).reshape(PR, IR)

    degp = _sc_degree(colm, pads)
    dis16, y1 = _sc_prep(degp, emb)
    s1 = _sc_layer(rowm, padg, colm, pads, y1)
    x1, y2 = _sc_mid(s1, dis16)
    s2 = _sc_layer(rowm, padg, colm, pads, y2)
    out_tbl = _sc_final(emb, x1, s2, dis16)
    return _sc_score(rowm, colm, padg, out_tbl)[:E]
